# baseline (speedup 1.0000x reference)
import jax
import jax.numpy as jnp
from jax import lax
from jax.experimental import pallas as pl
from jax.experimental.pallas import tpu as pltpu
from jax.experimental.pallas import tpu_sc as plsc

_RC = 4.6
_FACTOR = 13.605693122994 * 0.529177210903
_RC2 = _RC * _RC
_EPS_U = _RC2 * 1e-6
_LOG2E = 1.4426950408889634
_C2 = _RC2 * _LOG2E

_B, _N, _K = 24, 4096, 64
_NC, _NS = 2, 16
_NW = _NC * _NS
_AT = 256
_CPB = _N // _AT
_UNITS = _B * _CPB
_UPW = _UNITS // _NW
_NV = _AT // 16


def _tec_body(d_hbm, idx_hbm, q_hbm, out_hbm,
              table, dbuf0, dbuf1, ibuf0, ibuf1, part, accbuf,
              sd0, sd1, si0, si1):
    wid = lax.axis_index("s") * _NC + lax.axis_index("c")

    def _unit(i):
        g = wid * _UPW + i
        return g, g // _CPB, (g % _CPB) * _AT

    def _copies(i, dbuf, ibuf, sd, si):
        _, b, n0 = _unit(i)
        return (pltpu.make_async_copy(d_hbm.at[b, :, pl.ds(n0, _AT)], dbuf, sd),
                pltpu.make_async_copy(idx_hbm.at[b, :, pl.ds(n0, _AT)], ibuf, si))

    def _start(i, dbuf, ibuf, sd, si):
        hd, hi = _copies(i, dbuf, ibuf, sd, si)
        hd.start()
        hi.start()

    def _compute(i, dbuf, ibuf, sd, si):
        g, b, n0 = _unit(i)

        @pl.when((i == 0) | (n0 == 0))
        def _reload():
            pltpu.sync_copy(q_hbm.at[b], table)

        hd, hi = _copies(i, dbuf, ibuf, sd, si)
        hd.wait()
        hi.wait()

        zero = jnp.zeros((16,), jnp.float32)
        for nv in range(_NV):
            accbuf[pl.ds(16 * nv, 16)] = zero

        @plsc.parallel_loop(0, _K, 1, unroll=1)
        def _kbody(k):
            for nv in range(_NV):
                dv = dbuf[k, pl.ds(16 * nv, 16)]
                jv = ibuf[k, pl.ds(16 * nv, 16)]
                qj = plsc.load_gather(table, [jv])
                u = jnp.maximum(_RC2 - dv * dv, _EPS_U)
                r = 1.0 / (dv * u)
                f = jnp.exp(1.0 - _RC2 * (dv * r))
                plsc.addupdate(accbuf.at[pl.ds(16 * nv, 16)],
                               f * (qj * r) * u)

        unit = None
        for nv in range(_NV):
            contrib = (table[pl.ds(n0 + 16 * nv, 16)]
                       * accbuf[pl.ds(16 * nv, 16)])
            unit = contrib if unit is None else unit + contrib
        part[pl.ds(16 * i, 16)] = unit

    _start(0, dbuf0, ibuf0, sd0, si0)

    def _pair(p, carry):
        i0 = 2 * p
        _start(i0 + 1, dbuf1, ibuf1, sd1, si1)
        _compute(i0, dbuf0, ibuf0, sd0, si0)

        @pl.when(p + 1 < _UPW // 2)
        def _prefetch():
            _start(i0 + 2, dbuf0, ibuf0, sd0, si0)

        _compute(i0 + 1, dbuf1, ibuf1, sd1, si1)
        return carry

    lax.fori_loop(0, _UPW // 2, _pair, jnp.int32(0))
    pltpu.sync_copy(part, out_hbm.at[pl.ds(wid * (16 * _UPW), 16 * _UPW)])


def kernel(d_ij, charges, idx_j, pad_mask, energy):
    del pad_mask
    d_t = d_ij.transpose(0, 2, 1)
    idx_t = idx_j.astype(jnp.int32).transpose(0, 2, 1)
    mesh = plsc.VectorSubcoreMesh(
        core_axis_name="c", subcore_axis_name="s",
        num_cores=_NC, num_subcores=_NS)
    run = pl.kernel(
        _tec_body,
        out_type=jax.ShapeDtypeStruct((_UNITS * 16,), jnp.float32),
        mesh=mesh,
        compiler_params=pltpu.CompilerParams(needs_layout_passes=False),
        scratch_types=[
            pltpu.VMEM((_N,), jnp.float32),
            pltpu.VMEM((_K, _AT), jnp.float32),
            pltpu.VMEM((_K, _AT), jnp.float32),
            pltpu.VMEM((_K, _AT), jnp.int32),
            pltpu.VMEM((_K, _AT), jnp.int32),
            pltpu.VMEM((16 * _UPW,), jnp.float32),
            pltpu.VMEM((_AT,), jnp.float32),
            pltpu.SemaphoreType.DMA,
            pltpu.SemaphoreType.DMA,
            pltpu.SemaphoreType.DMA,
            pltpu.SemaphoreType.DMA,
        ],
    )
    parts = run(d_t, idx_t, charges)
    e_sr = parts.reshape(_B, _CPB * 16).sum(axis=-1)
    return energy - _FACTOR * e_sr


# speedup vs baseline: 1.2367x; 1.2367x over previous
import jax
import jax.numpy as jnp
from jax import lax
from jax.experimental import pallas as pl
from jax.experimental.pallas import tpu as pltpu
from jax.experimental.pallas import tpu_sc as plsc

_RC = 4.6
_FACTOR = 13.605693122994 * 0.529177210903
_RC2 = _RC * _RC
_EPS_U = _RC2 * 1e-6
import math
_FACTOR_E = _FACTOR * math.e

_B, _N, _K = 24, 4096, 64
_NC, _NS = 2, 16
_NW = _NC * _NS
_AT = 256
_CPB = _N // _AT
_UNITS = _B * _CPB
_UPW = _UNITS // _NW
_NV = _AT // 16


def _tec_body(d_hbm, idx_hbm, q_hbm, out_hbm,
              table, dbuf0, dbuf1, ibuf0, ibuf1, part,
              sd0, sd1, si0, si1):
    wid = lax.axis_index("s") * _NC + lax.axis_index("c")

    def _unit(i):
        g = wid * _UPW + i
        return g, g // _CPB, (g % _CPB) * _AT

    def _copies(i, dbuf, ibuf, sd, si):
        _, b, n0 = _unit(i)
        return (pltpu.make_async_copy(d_hbm.at[b, :, pl.ds(n0, _AT)], dbuf, sd),
                pltpu.make_async_copy(idx_hbm.at[b, :, pl.ds(n0, _AT)], ibuf, si))

    def _start(i, dbuf, ibuf, sd, si):
        hd, hi = _copies(i, dbuf, ibuf, sd, si)
        hd.start()
        hi.start()

    def _compute(i, dbuf, ibuf, sd, si):
        g, b, n0 = _unit(i)

        @pl.when((i == 0) | (n0 == 0))
        def _reload():
            pltpu.sync_copy(q_hbm.at[b], table)

        hd, hi = _copies(i, dbuf, ibuf, sd, si)
        hd.wait()
        hi.wait()

        def _kbody(k, accs):
            out = []
            for nv in range(_NV):
                dv = dbuf[k, pl.ds(16 * nv, 16)]
                jv = ibuf[k, pl.ds(16 * nv, 16)]
                qj = plsc.load_gather(table, [jv])
                u = jnp.maximum(_RC2 - dv * dv, _EPS_U)
                r = 1.0 / (dv * u)
                f = jnp.exp((-_RC2) * (dv * r))
                out.append(accs[nv] + f * (qj * r) * u)
            return tuple(out)

        accs = lax.fori_loop(
            0, _K, _kbody, tuple(jnp.zeros((16,), jnp.float32)
                                 for _ in range(_NV)))
        unit = None
        for nv in range(_NV):
            contrib = table[pl.ds(n0 + 16 * nv, 16)] * accs[nv]
            unit = contrib if unit is None else unit + contrib
        part[pl.ds(16 * i, 16)] = unit

    _start(0, dbuf0, ibuf0, sd0, si0)

    def _pair(p, carry):
        i0 = 2 * p
        _start(i0 + 1, dbuf1, ibuf1, sd1, si1)
        _compute(i0, dbuf0, ibuf0, sd0, si0)

        @pl.when(p + 1 < _UPW // 2)
        def _prefetch():
            _start(i0 + 2, dbuf0, ibuf0, sd0, si0)

        _compute(i0 + 1, dbuf1, ibuf1, sd1, si1)
        return carry

    lax.fori_loop(0, _UPW // 2, _pair, jnp.int32(0))
    pltpu.sync_copy(part, out_hbm.at[pl.ds(wid * (16 * _UPW), 16 * _UPW)])


def kernel(d_ij, charges, idx_j, pad_mask, energy):
    del pad_mask
    d_t = d_ij.transpose(0, 2, 1)
    idx_t = idx_j.astype(jnp.int32).transpose(0, 2, 1)
    mesh = plsc.VectorSubcoreMesh(
        core_axis_name="c", subcore_axis_name="s",
        num_cores=_NC, num_subcores=_NS)
    run = pl.kernel(
        _tec_body,
        out_type=jax.ShapeDtypeStruct((_UNITS * 16,), jnp.float32),
        mesh=mesh,
        compiler_params=pltpu.CompilerParams(needs_layout_passes=False),
        scratch_types=[
            pltpu.VMEM((_N,), jnp.float32),
            pltpu.VMEM((_K, _AT), jnp.float32),
            pltpu.VMEM((_K, _AT), jnp.float32),
            pltpu.VMEM((_K, _AT), jnp.int32),
            pltpu.VMEM((_K, _AT), jnp.int32),
            pltpu.VMEM((16 * _UPW,), jnp.float32),
            pltpu.SemaphoreType.DMA,
            pltpu.SemaphoreType.DMA,
            pltpu.SemaphoreType.DMA,
            pltpu.SemaphoreType.DMA,
        ],
    )
    parts = run(d_t, idx_t, charges)
    e_sr = parts.reshape(_B, _CPB * 16).sum(axis=-1)
    return energy - _FACTOR_E * e_sr
